# Initial kernel scaffold; baseline (speedup 1.0000x reference)
import jax
import jax.numpy as jnp
from jax.experimental import pallas as pl


def kernel(x, edge_index, batch, enc_W1, enc_b1, enc_W2, enc_b2, W1l, b1l, W1r, W2l, b2l, W2r, W3l, b3l, W3r, dec_W1, dec_b1, dec_W2, dec_b2):
    raise NotImplementedError("write your pallas kernel here")


# speedup vs baseline: 6.4292x; 6.4292x over previous
import functools

import jax
import jax.numpy as jnp
from jax import lax
from jax.experimental import pallas as pl
from jax.experimental.pallas import tpu as pltpu
from jax.experimental.pallas import tpu_sc as plsc

N_NODES = 50000
N_EDGES = 800000
D_IN = 128
D_HID = 64
HALF = 32
N_GRAPHS = 64

N_TILES = 16
N_CORES = 2
ROWS_PER_TILE = 3136
N_PAD = N_TILES * ROWS_PER_TILE
E_PER_TILE = N_EDGES // N_TILES
E_CHUNK = 400
N_CHUNKS = E_PER_TILE // E_CHUNK

NB = 2000
N_BLOCKS = N_NODES // NB


def _sc_agg_body(ha, hb, src, dst, zrows, zcnt, ones_h,
                 suma, sumb, cnt,
                 idx_v, dst_v, rows_v, ones_v, stage_v, acc_sh, cnt_sh, sem):
    cid = lax.axis_index("c")
    sid = lax.axis_index("s")
    base = sid * ROWS_PER_TILE

    pltpu.sync_copy(zrows.at[pl.ds(base, ROWS_PER_TILE)],
                    acc_sh.at[pl.ds(base, ROWS_PER_TILE)])

    @pl.when(cid == 0)
    def _():
        pltpu.sync_copy(zcnt.at[pl.ds(base, ROWS_PER_TILE)], stage_v)
        pltpu.sync_copy(stage_v, cnt_sh.at[pl.ds(base, ROWS_PER_TILE)])
        pltpu.sync_copy(ones_h, ones_v)

    plsc.subcore_barrier()

    ebase0 = sid * E_PER_TILE

    def chunk(c, carry):
        eb = ebase0 + c * E_CHUNK
        pltpu.sync_copy(src.at[pl.ds(eb, E_CHUNK)], idx_v)
        pltpu.sync_copy(dst.at[pl.ds(eb, E_CHUNK)], dst_v)

        @pl.when(cid == 0)
        def _():
            pltpu.async_copy(ha.at[idx_v], rows_v, sem).wait()

        @pl.when(cid == 1)
        def _():
            pltpu.async_copy(hb.at[idx_v], rows_v, sem).wait()

        pltpu.sync_copy(rows_v, acc_sh.at[dst_v], add=True)

        @pl.when(cid == 0)
        def _():
            pltpu.sync_copy(ones_v, cnt_sh.at[dst_v], add=True)

        return carry

    lax.fori_loop(0, N_CHUNKS, chunk, 0)
    plsc.subcore_barrier()

    @pl.when(cid == 0)
    def _():
        pltpu.sync_copy(acc_sh.at[pl.ds(base, ROWS_PER_TILE)],
                        suma.at[pl.ds(base, ROWS_PER_TILE)])
        pltpu.sync_copy(cnt_sh.at[pl.ds(base, ROWS_PER_TILE)], stage_v)
        pltpu.sync_copy(stage_v, cnt.at[pl.ds(base, ROWS_PER_TILE)])

    @pl.when(cid == 1)
    def _():
        pltpu.sync_copy(acc_sh.at[pl.ds(base, ROWS_PER_TILE)],
                        sumb.at[pl.ds(base, ROWS_PER_TILE)])


_sc_agg = functools.partial(
    pl.kernel,
    out_type=(
        jax.ShapeDtypeStruct((N_PAD, HALF), jnp.float32),
        jax.ShapeDtypeStruct((N_PAD, HALF), jnp.float32),
        jax.ShapeDtypeStruct((N_PAD,), jnp.float32),
    ),
    mesh=plsc.VectorSubcoreMesh(core_axis_name="c", subcore_axis_name="s",
                                num_cores=N_CORES, num_subcores=N_TILES),
    scratch_types=[
        pltpu.VMEM((E_CHUNK,), jnp.int32),
        pltpu.VMEM((E_CHUNK,), jnp.int32),
        pltpu.VMEM((E_CHUNK, HALF), jnp.float32),
        pltpu.VMEM((E_CHUNK,), jnp.float32),
        pltpu.VMEM((ROWS_PER_TILE,), jnp.float32),
        pltpu.VMEM_SHARED((N_PAD, HALF), jnp.float32),
        pltpu.VMEM_SHARED((N_PAD,), jnp.float32),
        pltpu.SemaphoreType.DMA,
    ],
    compiler_params=pltpu.CompilerParams(use_tc_tiling_on_sc=False),
)(_sc_agg_body)


def _enc_body(x_ref, w1, b1, w2, b2, oa, ob):
    h = jnp.dot(x_ref[...], w1[...], preferred_element_type=jnp.float32)
    h = jnp.maximum(h + b1[...], 0.0)
    h = jnp.dot(h, w2[...], preferred_element_type=jnp.float32)
    h = jnp.maximum(h + b2[...], 0.0)
    oa[...] = h[:, :HALF]
    ob[...] = h[:, HALF:]


def _encoder(x, w1, b1, w2, b2):
    return pl.pallas_call(
        _enc_body,
        grid=(N_BLOCKS,),
        in_specs=[
            pl.BlockSpec((NB, D_IN), lambda i: (i, 0)),
            pl.BlockSpec((D_IN, HALF), lambda i: (0, 0)),
            pl.BlockSpec((1, HALF), lambda i: (0, 0)),
            pl.BlockSpec((HALF, D_HID), lambda i: (0, 0)),
            pl.BlockSpec((1, D_HID), lambda i: (0, 0)),
        ],
        out_specs=[
            pl.BlockSpec((NB, HALF), lambda i: (i, 0)),
            pl.BlockSpec((NB, HALF), lambda i: (i, 0)),
        ],
        out_shape=[
            jax.ShapeDtypeStruct((N_PAD, HALF), jnp.float32),
            jax.ShapeDtypeStruct((N_PAD, HALF), jnp.float32),
        ],
    )(x, w1, b1, w2, b2)


def _sage_mix(sa, sb, cnt, ha, hb, wl, bl, wr):
    r = 1.0 / jnp.maximum(cnt, 1.0)
    h = (jnp.dot(sa * r, wl[:HALF], preferred_element_type=jnp.float32)
         + jnp.dot(sb * r, wl[HALF:], preferred_element_type=jnp.float32)
         + bl
         + jnp.dot(ha, wr[:HALF], preferred_element_type=jnp.float32)
         + jnp.dot(hb, wr[HALF:], preferred_element_type=jnp.float32))
    return h


def _layer_body(sa, sb, cnt, ha, hb, wl, bl, wr, oa, ob):
    h = jnp.maximum(
        _sage_mix(sa[...], sb[...], cnt[...], ha[...], hb[...],
                  wl[...], bl[...], wr[...]), 0.0)
    oa[...] = h[:, :HALF]
    ob[...] = h[:, HALF:]


def _layer(sa, sb, cnt2, ha, hb, wl, bl2, wr):
    return pl.pallas_call(
        _layer_body,
        grid=(N_BLOCKS,),
        in_specs=[
            pl.BlockSpec((NB, HALF), lambda i: (i, 0)),
            pl.BlockSpec((NB, HALF), lambda i: (i, 0)),
            pl.BlockSpec((NB, 1), lambda i: (i, 0)),
            pl.BlockSpec((NB, HALF), lambda i: (i, 0)),
            pl.BlockSpec((NB, HALF), lambda i: (i, 0)),
            pl.BlockSpec((D_HID, D_HID), lambda i: (0, 0)),
            pl.BlockSpec((1, D_HID), lambda i: (0, 0)),
            pl.BlockSpec((D_HID, D_HID), lambda i: (0, 0)),
        ],
        out_specs=[
            pl.BlockSpec((NB, HALF), lambda i: (i, 0)),
            pl.BlockSpec((NB, HALF), lambda i: (i, 0)),
        ],
        out_shape=[
            jax.ShapeDtypeStruct((N_PAD, HALF), jnp.float32),
            jax.ShapeDtypeStruct((N_PAD, HALF), jnp.float32),
        ],
    )(sa, sb, cnt2, ha, hb, wl, bl2, wr)


def _final_body(sa, sb, cnt, ha, hb, batch_r, w3l, b3l, w3r,
                dw1, db1, dw2, db2, out_ref):
    i = pl.program_id(0)
    h = jnp.maximum(
        _sage_mix(sa[...], sb[...], cnt[...], ha[...], hb[...],
                  w3l[...], b3l[...], w3r[...]), 0.0)
    d = jnp.maximum(
        jnp.dot(h, dw1[...], preferred_element_type=jnp.float32) + db1[...],
        0.0)
    o = jnp.dot(d, dw2[...], preferred_element_type=jnp.float32) + db2[...]

    b = batch_r[...]

    @pl.when(i == 0)
    def _():
        out_ref[...] = jnp.full((N_GRAPHS, 48), -jnp.inf, jnp.float32)

    bmin = jnp.min(b)
    bmax = jnp.max(b)
    for g in range(N_GRAPHS):
        @pl.when((g >= bmin) & (g <= bmax))
        def _():
            m = jnp.max(jnp.where(b == g, o, -jnp.inf), axis=0, keepdims=True)
            out_ref[pl.ds(g, 1), :] = jnp.maximum(out_ref[pl.ds(g, 1), :], m)


def _final(sa, sb, cnt2, ha, hb, batch2, w3l, b3l2, w3r, dw1, db12, dw2, db22):
    return pl.pallas_call(
        _final_body,
        grid=(N_BLOCKS,),
        in_specs=[
            pl.BlockSpec((NB, HALF), lambda i: (i, 0)),
            pl.BlockSpec((NB, HALF), lambda i: (i, 0)),
            pl.BlockSpec((NB, 1), lambda i: (i, 0)),
            pl.BlockSpec((NB, HALF), lambda i: (i, 0)),
            pl.BlockSpec((NB, HALF), lambda i: (i, 0)),
            pl.BlockSpec((NB, 1), lambda i: (i, 0)),
            pl.BlockSpec((D_HID, D_HID), lambda i: (0, 0)),
            pl.BlockSpec((1, D_HID), lambda i: (0, 0)),
            pl.BlockSpec((D_HID, D_HID), lambda i: (0, 0)),
            pl.BlockSpec((D_HID, HALF), lambda i: (0, 0)),
            pl.BlockSpec((1, HALF), lambda i: (0, 0)),
            pl.BlockSpec((HALF, 48), lambda i: (0, 0)),
            pl.BlockSpec((1, 48), lambda i: (0, 0)),
        ],
        out_specs=pl.BlockSpec((N_GRAPHS, 48), lambda i: (0, 0)),
        out_shape=jax.ShapeDtypeStruct((N_GRAPHS, 48), jnp.float32),
    )(sa, sb, cnt2, ha, hb, batch2, w3l, b3l2, w3r, dw1, db12, dw2, db22)


def kernel(x, edge_index, batch, enc_W1, enc_b1, enc_W2, enc_b2,
           W1l, b1l, W1r, W2l, b2l, W2r, W3l, b3l, W3r,
           dec_W1, dec_b1, dec_W2, dec_b2):
    src = edge_index[0]
    dst = edge_index[1]
    zrows = jnp.zeros((N_PAD, HALF), jnp.float32)
    zcnt = jnp.zeros((N_PAD,), jnp.float32)
    ones_h = jnp.ones((E_CHUNK,), jnp.float32)
    batch2 = batch.reshape(-1, 1)

    ha, hb = _encoder(x, enc_W1, enc_b1.reshape(1, -1),
                      enc_W2, enc_b2.reshape(1, -1))

    sa, sb, cnt = _sc_agg(ha, hb, src, dst, zrows, zcnt, ones_h)
    cnt2 = cnt.reshape(-1, 1)
    ha, hb = _layer(sa, sb, cnt2, ha, hb, W1l, b1l.reshape(1, -1), W1r)

    sa, sb, cnt = _sc_agg(ha, hb, src, dst, zrows, zcnt, ones_h)
    cnt2 = cnt.reshape(-1, 1)
    ha, hb = _layer(sa, sb, cnt2, ha, hb, W2l, b2l.reshape(1, -1), W2r)

    sa, sb, cnt = _sc_agg(ha, hb, src, dst, zrows, zcnt, ones_h)
    cnt2 = cnt.reshape(-1, 1)
    pooled = _final(sa, sb, cnt2, ha, hb, batch2,
                    W3l, b3l.reshape(1, -1), W3r,
                    dec_W1, dec_b1.reshape(1, -1),
                    dec_W2, dec_b2.reshape(1, -1))

    return pooled.reshape(-1, 12)


# speedup vs baseline: 8.7673x; 1.3637x over previous
import functools

import jax
import jax.numpy as jnp
from jax import lax
from jax.experimental import pallas as pl
from jax.experimental.pallas import tpu as pltpu
from jax.experimental.pallas import tpu_sc as plsc

N_NODES = 50000
N_EDGES = 800000
D_IN = 128
D_HID = 64
HALF = 32
N_GRAPHS = 64

N_TILES = 16
N_CORES = 2
ROWS_PER_TILE = N_NODES // N_TILES
E_PER_TILE = N_EDGES // N_TILES
E_CHUNK = 400
N_CHUNKS = E_PER_TILE // E_CHUNK
N_PAIRS = (N_CHUNKS + 1) // 2

N_CPAD = 50048
CNT_PER_TILE = N_CPAD // N_TILES
E_PER_CTILE = N_EDGES // (N_CORES * N_TILES)
EC_CHUNK = 1000
NC_CHUNKS = E_PER_CTILE // EC_CHUNK

NB = 2000
N_BLOCKS = N_NODES // NB

_MESH = plsc.VectorSubcoreMesh(core_axis_name="c", subcore_axis_name="s",
                               num_cores=N_CORES, num_subcores=N_TILES)


def _sc_counts_body(dst, zcnt, ones_h, cnt0, cnt1,
                    dst_v, ones_v, stage_v, cnt_sh):
    cid = lax.axis_index("c")
    sid = lax.axis_index("s")
    base = sid * CNT_PER_TILE

    pltpu.sync_copy(zcnt.at[pl.ds(base, CNT_PER_TILE)], stage_v)
    pltpu.sync_copy(stage_v, cnt_sh.at[pl.ds(base, CNT_PER_TILE)])
    pltpu.sync_copy(ones_h, ones_v)
    plsc.subcore_barrier()

    ebase0 = (cid * N_TILES + sid) * E_PER_CTILE

    def chunk(c, carry):
        eb = ebase0 + c * EC_CHUNK
        pltpu.sync_copy(dst.at[pl.ds(eb, EC_CHUNK)], dst_v)
        pltpu.sync_copy(ones_v, cnt_sh.at[dst_v], add=True)
        return carry

    lax.fori_loop(0, NC_CHUNKS, chunk, 0)
    plsc.subcore_barrier()

    pltpu.sync_copy(cnt_sh.at[pl.ds(base, CNT_PER_TILE)], stage_v)

    @pl.when(cid == 0)
    def _():
        pltpu.sync_copy(stage_v, cnt0.at[pl.ds(base, CNT_PER_TILE)])

    @pl.when(cid == 1)
    def _():
        pltpu.sync_copy(stage_v, cnt1.at[pl.ds(base, CNT_PER_TILE)])


_sc_counts = functools.partial(
    pl.kernel,
    out_type=(
        jax.ShapeDtypeStruct((N_CPAD,), jnp.float32),
        jax.ShapeDtypeStruct((N_CPAD,), jnp.float32),
    ),
    mesh=_MESH,
    scratch_types=[
        pltpu.VMEM((EC_CHUNK,), jnp.int32),
        pltpu.VMEM((EC_CHUNK,), jnp.float32),
        pltpu.VMEM((CNT_PER_TILE,), jnp.float32),
        pltpu.VMEM_SHARED((N_CPAD,), jnp.float32),
    ],
    compiler_params=pltpu.CompilerParams(use_tc_tiling_on_sc=False),
)(_sc_counts_body)


def _sc_agg_body(ha, hb, src, dst, zrows,
                 suma, sumb,
                 idx_v0, idx_v1, dst_v0, dst_v1, rows_v0, rows_v1,
                 acc_sh, sem0, sem1):
    cid = lax.axis_index("c")
    sid = lax.axis_index("s")
    base = sid * ROWS_PER_TILE

    pltpu.sync_copy(zrows.at[pl.ds(base, ROWS_PER_TILE)],
                    acc_sh.at[pl.ds(base, ROWS_PER_TILE)])
    plsc.subcore_barrier()

    ebase0 = sid * E_PER_TILE

    def load_idx(c, idx_v, dst_v):
        eb = ebase0 + c * E_CHUNK
        pltpu.sync_copy(src.at[pl.ds(eb, E_CHUNK)], idx_v)
        pltpu.sync_copy(dst.at[pl.ds(eb, E_CHUNK)], dst_v)

    def start_gather(idx_v, rows_v, sem):
        @pl.when(cid == 0)
        def _():
            pltpu.async_copy(ha.at[idx_v], rows_v, sem)

        @pl.when(cid == 1)
        def _():
            pltpu.async_copy(hb.at[idx_v], rows_v, sem)

    def wait_gather(idx_v, rows_v, sem):
        @pl.when(cid == 0)
        def _():
            pltpu.make_async_copy(ha.at[idx_v], rows_v, sem).wait()

        @pl.when(cid == 1)
        def _():
            pltpu.make_async_copy(hb.at[idx_v], rows_v, sem).wait()

    load_idx(0, idx_v0, dst_v0)
    start_gather(idx_v0, rows_v0, sem0)
    load_idx(1, idx_v1, dst_v1)
    start_gather(idx_v1, rows_v1, sem1)

    def step(c, idx_v, dst_v, rows_v, sem):
        @pl.when(c < N_CHUNKS)
        def _():
            wait_gather(idx_v, rows_v, sem)
            pltpu.sync_copy(rows_v, acc_sh.at[dst_v], add=True)

            @pl.when(c + 2 < N_CHUNKS)
            def _():
                load_idx(c + 2, idx_v, dst_v)
                start_gather(idx_v, rows_v, sem)

    def pair(i, carry):
        step(2 * i, idx_v0, dst_v0, rows_v0, sem0)
        step(2 * i + 1, idx_v1, dst_v1, rows_v1, sem1)
        return carry

    lax.fori_loop(0, N_PAIRS, pair, 0)
    plsc.subcore_barrier()

    @pl.when(cid == 0)
    def _():
        pltpu.sync_copy(acc_sh.at[pl.ds(base, ROWS_PER_TILE)],
                        suma.at[pl.ds(base, ROWS_PER_TILE)])

    @pl.when(cid == 1)
    def _():
        pltpu.sync_copy(acc_sh.at[pl.ds(base, ROWS_PER_TILE)],
                        sumb.at[pl.ds(base, ROWS_PER_TILE)])


_sc_agg = functools.partial(
    pl.kernel,
    out_type=(
        jax.ShapeDtypeStruct((N_NODES, HALF), jnp.float32),
        jax.ShapeDtypeStruct((N_NODES, HALF), jnp.float32),
    ),
    mesh=_MESH,
    scratch_types=[
        pltpu.VMEM((E_CHUNK,), jnp.int32),
        pltpu.VMEM((E_CHUNK,), jnp.int32),
        pltpu.VMEM((E_CHUNK,), jnp.int32),
        pltpu.VMEM((E_CHUNK,), jnp.int32),
        pltpu.VMEM((E_CHUNK, HALF), jnp.float32),
        pltpu.VMEM((E_CHUNK, HALF), jnp.float32),
        pltpu.VMEM_SHARED((N_NODES, HALF), jnp.float32),
        pltpu.SemaphoreType.DMA,
        pltpu.SemaphoreType.DMA,
    ],
    compiler_params=pltpu.CompilerParams(use_tc_tiling_on_sc=False),
)(_sc_agg_body)


def _enc_body(x_ref, w1, b1, w2, b2, oa, ob):
    h = jnp.dot(x_ref[...], w1[...], preferred_element_type=jnp.float32)
    h = jnp.maximum(h + b1[...], 0.0)
    h = jnp.dot(h, w2[...], preferred_element_type=jnp.float32)
    h = jnp.maximum(h + b2[...], 0.0)
    oa[...] = h[:, :HALF]
    ob[...] = h[:, HALF:]


def _encoder(x, w1, b1, w2, b2):
    return pl.pallas_call(
        _enc_body,
        grid=(N_BLOCKS,),
        in_specs=[
            pl.BlockSpec((NB, D_IN), lambda i: (i, 0)),
            pl.BlockSpec((D_IN, HALF), lambda i: (0, 0)),
            pl.BlockSpec((1, HALF), lambda i: (0, 0)),
            pl.BlockSpec((HALF, D_HID), lambda i: (0, 0)),
            pl.BlockSpec((1, D_HID), lambda i: (0, 0)),
        ],
        out_specs=[
            pl.BlockSpec((NB, HALF), lambda i: (i, 0)),
            pl.BlockSpec((NB, HALF), lambda i: (i, 0)),
        ],
        out_shape=[
            jax.ShapeDtypeStruct((N_NODES, HALF), jnp.float32),
            jax.ShapeDtypeStruct((N_NODES, HALF), jnp.float32),
        ],
    )(x, w1, b1, w2, b2)


def _sage_mix(sa, sb, c0, c1, ha, hb, wl, bl, wr):
    r = 1.0 / jnp.maximum(c0 + c1, 1.0)
    h = (jnp.dot(sa * r, wl[:HALF], preferred_element_type=jnp.float32)
         + jnp.dot(sb * r, wl[HALF:], preferred_element_type=jnp.float32)
         + bl
         + jnp.dot(ha, wr[:HALF], preferred_element_type=jnp.float32)
         + jnp.dot(hb, wr[HALF:], preferred_element_type=jnp.float32))
    return h


def _layer_body(sa, sb, c0, c1, ha, hb, wl, bl, wr, oa, ob):
    h = jnp.maximum(
        _sage_mix(sa[...], sb[...], c0[...], c1[...], ha[...], hb[...],
                  wl[...], bl[...], wr[...]), 0.0)
    oa[...] = h[:, :HALF]
    ob[...] = h[:, HALF:]


def _layer(sa, sb, c02, c12, ha, hb, wl, bl2, wr):
    return pl.pallas_call(
        _layer_body,
        grid=(N_BLOCKS,),
        in_specs=[
            pl.BlockSpec((NB, HALF), lambda i: (i, 0)),
            pl.BlockSpec((NB, HALF), lambda i: (i, 0)),
            pl.BlockSpec((NB, 1), lambda i: (i, 0)),
            pl.BlockSpec((NB, 1), lambda i: (i, 0)),
            pl.BlockSpec((NB, HALF), lambda i: (i, 0)),
            pl.BlockSpec((NB, HALF), lambda i: (i, 0)),
            pl.BlockSpec((D_HID, D_HID), lambda i: (0, 0)),
            pl.BlockSpec((1, D_HID), lambda i: (0, 0)),
            pl.BlockSpec((D_HID, D_HID), lambda i: (0, 0)),
        ],
        out_specs=[
            pl.BlockSpec((NB, HALF), lambda i: (i, 0)),
            pl.BlockSpec((NB, HALF), lambda i: (i, 0)),
        ],
        out_shape=[
            jax.ShapeDtypeStruct((N_NODES, HALF), jnp.float32),
            jax.ShapeDtypeStruct((N_NODES, HALF), jnp.float32),
        ],
    )(sa, sb, c02, c12, ha, hb, wl, bl2, wr)


def _final_body(sa, sb, c0, c1, ha, hb, batch_r, w3l, b3l, w3r,
                dw1, db1, dw2, db2, out_ref):
    i = pl.program_id(0)
    h = jnp.maximum(
        _sage_mix(sa[...], sb[...], c0[...], c1[...], ha[...], hb[...],
                  w3l[...], b3l[...], w3r[...]), 0.0)
    d = jnp.maximum(
        jnp.dot(h, dw1[...], preferred_element_type=jnp.float32) + db1[...],
        0.0)
    o = jnp.dot(d, dw2[...], preferred_element_type=jnp.float32) + db2[...]

    b = batch_r[...]

    @pl.when(i == 0)
    def _():
        out_ref[...] = jnp.full((N_GRAPHS, 48), -jnp.inf, jnp.float32)

    bmin = jnp.min(b)
    bmax = jnp.max(b)
    for g in range(N_GRAPHS):
        @pl.when((g >= bmin) & (g <= bmax))
        def _():
            m = jnp.max(jnp.where(b == g, o, -jnp.inf), axis=0, keepdims=True)
            out_ref[pl.ds(g, 1), :] = jnp.maximum(out_ref[pl.ds(g, 1), :], m)


def _final(sa, sb, c02, c12, ha, hb, batch2, w3l, b3l2, w3r,
           dw1, db12, dw2, db22):
    return pl.pallas_call(
        _final_body,
        grid=(N_BLOCKS,),
        in_specs=[
            pl.BlockSpec((NB, HALF), lambda i: (i, 0)),
            pl.BlockSpec((NB, HALF), lambda i: (i, 0)),
            pl.BlockSpec((NB, 1), lambda i: (i, 0)),
            pl.BlockSpec((NB, 1), lambda i: (i, 0)),
            pl.BlockSpec((NB, HALF), lambda i: (i, 0)),
            pl.BlockSpec((NB, HALF), lambda i: (i, 0)),
            pl.BlockSpec((NB, 1), lambda i: (i, 0)),
            pl.BlockSpec((D_HID, D_HID), lambda i: (0, 0)),
            pl.BlockSpec((1, D_HID), lambda i: (0, 0)),
            pl.BlockSpec((D_HID, D_HID), lambda i: (0, 0)),
            pl.BlockSpec((D_HID, HALF), lambda i: (0, 0)),
            pl.BlockSpec((1, HALF), lambda i: (0, 0)),
            pl.BlockSpec((HALF, 48), lambda i: (0, 0)),
            pl.BlockSpec((1, 48), lambda i: (0, 0)),
        ],
        out_specs=pl.BlockSpec((N_GRAPHS, 48), lambda i: (0, 0)),
        out_shape=jax.ShapeDtypeStruct((N_GRAPHS, 48), jnp.float32),
    )(sa, sb, c02, c12, ha, hb, batch2, w3l, b3l2, w3r, dw1, db12, dw2, db22)


def kernel(x, edge_index, batch, enc_W1, enc_b1, enc_W2, enc_b2,
           W1l, b1l, W1r, W2l, b2l, W2r, W3l, b3l, W3r,
           dec_W1, dec_b1, dec_W2, dec_b2):
    src = edge_index[0]
    dst = edge_index[1]
    zrows = jnp.zeros((N_NODES, HALF), jnp.float32)
    zcnt = jnp.zeros((N_CPAD,), jnp.float32)
    ones_h = jnp.ones((EC_CHUNK,), jnp.float32)
    batch2 = batch.reshape(-1, 1)

    cnt0, cnt1 = _sc_counts(dst, zcnt, ones_h)
    c02 = cnt0[:N_NODES].reshape(-1, 1)
    c12 = cnt1[:N_NODES].reshape(-1, 1)

    ha, hb = _encoder(x, enc_W1, enc_b1.reshape(1, -1),
                      enc_W2, enc_b2.reshape(1, -1))

    sa, sb = _sc_agg(ha, hb, src, dst, zrows)
    ha, hb = _layer(sa, sb, c02, c12, ha, hb, W1l, b1l.reshape(1, -1), W1r)

    sa, sb = _sc_agg(ha, hb, src, dst, zrows)
    ha, hb = _layer(sa, sb, c02, c12, ha, hb, W2l, b2l.reshape(1, -1), W2r)

    sa, sb = _sc_agg(ha, hb, src, dst, zrows)
    pooled = _final(sa, sb, c02, c12, ha, hb, batch2,
                    W3l, b3l.reshape(1, -1), W3r,
                    dec_W1, dec_b1.reshape(1, -1),
                    dec_W2, dec_b2.reshape(1, -1))

    return pooled.reshape(-1, 12)
